# Initial kernel scaffold; baseline (speedup 1.0000x reference)
import jax
import jax.numpy as jnp
from jax.experimental import pallas as pl


def kernel(visual_features, texts, embed_table, W_proj, b_proj, image_token_id, pad_token_id):
    raise NotImplementedError("write your pallas kernel here")


# speedup vs baseline: 2.0297x; 2.0297x over previous
import functools

import jax
import jax.numpy as jnp
from jax import lax
from jax.experimental import pallas as pl
from jax.experimental.pallas import tpu as pltpu
from jax.experimental.pallas import tpu_sc as plsc

_NC = 2
_NS = 16
_NW = _NC * _NS
_K = 32

_B = 8
_S = 2048
_D = 2048
_NV = 512
_MAX_LEN = 2234
_R = _B * _MAX_LEN
_CAP_T = _B * _S
_DUMP = _R


def _fusion_indices(texts, image_token_id, pad_token_id):
    pos = jnp.arange(_S, dtype=jnp.int32)
    toks = texts.astype(jnp.int32)
    L = jnp.sum((toks != pad_token_id).astype(jnp.int32), axis=1)
    valid = pos[None, :] < L[:, None]
    img = (toks == image_token_id) & valid
    n_img = jnp.sum(img.astype(jnp.int32), axis=1)
    vpt = _NV // jnp.maximum(n_img, 1)
    before = jnp.cumsum(img.astype(jnp.int32), axis=1) - img.astype(jnp.int32)
    out_text = pos[None, :] + before * (vpt[:, None] - 1)
    text_act = valid & (~img)
    row_base = (jnp.arange(_B, dtype=jnp.int32) * _MAX_LEN)[:, None]

    dst_t = jnp.where(text_act, row_base + out_text, _DUMP).reshape(-1)
    tok_t = jnp.where(text_act, toks, 0).reshape(-1)
    perm_t = jnp.argsort(jnp.where(text_act.reshape(-1), 0, 1), stable=True)
    dst_t = dst_t[perm_t]
    tok_t = tok_t[perm_t]
    n_text = jnp.sum(text_act.astype(jnp.int32))

    img_pos = jnp.sort(jnp.where(img, pos[None, :], _S), axis=1)
    vidx = jnp.arange(_NV, dtype=jnp.int32)
    bi = vidx[None, :] // vpt[:, None]
    w = vidx[None, :] - bi * vpt[:, None]
    p_b = jnp.take_along_axis(img_pos, jnp.minimum(bi, _S - 1), axis=1)
    out_vis = p_b + bi * (vpt[:, None] - 1) + w
    vis_act = vidx[None, :] < (n_img * vpt)[:, None]
    dst_v = jnp.where(vis_act, row_base + out_vis, _DUMP).reshape(-1)

    length = L - n_img + n_img * vpt
    cols = jnp.arange(_MAX_LEN, dtype=jnp.int32)
    padm = cols[None, :] >= length[:, None]
    dst_p = jnp.where(padm, row_base + cols[None, :], _DUMP).reshape(-1)
    perm_p = jnp.argsort(jnp.where(padm.reshape(-1), 0, 1), stable=True)
    dst_p = dst_p[perm_p]
    n_pad = jnp.sum(padm.astype(jnp.int32))

    counts = jnp.zeros((16,), jnp.int32).at[0].set(n_text).at[1].set(n_pad)
    attn = ~padm
    return tok_t, dst_t, dst_v, dst_p, counts, attn


def _project(vf_flat, w_proj, b_proj):
    m, k = vf_flat.shape
    n = w_proj.shape[1]
    bm = 512

    def body(a_ref, w_ref, b_ref, o_ref):
        o_ref[...] = (
            jnp.dot(a_ref[...], w_ref[...], preferred_element_type=jnp.float32)
            + b_ref[...]
        )

    return pl.pallas_call(
        body,
        grid=(m // bm,),
        in_specs=[
            pl.BlockSpec((bm, k), lambda i: (i, 0)),
            pl.BlockSpec((k, n), lambda i: (0, 0)),
            pl.BlockSpec((n,), lambda i: (0,)),
        ],
        out_specs=pl.BlockSpec((bm, n), lambda i: (i, 0)),
        out_shape=jax.ShapeDtypeStruct((m, n), jnp.float32),
    )(vf_flat, w_proj, b_proj)


def _sc_fuse(embed, proj, tok_t, dst_t, dst_v, dst_p, counts, zrows):
    mesh = plsc.VectorSubcoreMesh(
        core_axis_name="c", subcore_axis_name="s", num_cores=_NC, num_subcores=_NS
    )

    @functools.partial(
        pl.kernel,
        out_type=jax.ShapeDtypeStruct((_R + 8, _D), jnp.float32),
        mesh=mesh,
        scratch_types=[
            pltpu.VMEM((16,), jnp.int32),
            pltpu.VMEM((_K,), jnp.int32),
            pltpu.VMEM((_K,), jnp.int32),
            pltpu.VMEM((_K, _D), jnp.float32),
            pltpu.SemaphoreType.DMA,
            pltpu.SemaphoreType.DMA,
        ],
    )
    def k(embed_h, proj_h, tok_h, dstt_h, dstv_h, dstp_h, cnt_h, z_h, out_h,
          cnt_v, idx_v, dst_v_ref, buf_v, sem_g, sem_s):
        wid = lax.axis_index("s") * _NC + lax.axis_index("c")
        pltpu.sync_copy(cnt_h, cnt_v)
        cv = cnt_v[...]
        n_text = cv[0]
        n_pad = cv[1]

        def trips(ncount):
            return jnp.maximum(0, (ncount - wid * _K + (_NW * _K - 1)) // (_NW * _K))

        def text_body(i, carry):
            base = (wid + i * _NW) * _K
            pltpu.sync_copy(tok_h.at[pl.ds(base, _K)], idx_v)
            pltpu.sync_copy(dstt_h.at[pl.ds(base, _K)], dst_v_ref)
            pltpu.async_copy(embed_h.at[idx_v], buf_v, sem_g).wait()
            pltpu.async_copy(buf_v, out_h.at[dst_v_ref], sem_s).wait()
            return carry

        lax.fori_loop(0, trips(n_text), text_body, 0)

        def vis_body(i, carry):
            base = (wid + i * _NW) * _K
            pltpu.sync_copy(dstv_h.at[pl.ds(base, _K)], dst_v_ref)
            pltpu.sync_copy(proj_h.at[pl.ds(base, _K)], buf_v)
            pltpu.async_copy(buf_v, out_h.at[dst_v_ref], sem_s).wait()
            return carry

        lax.fori_loop(0, (_B * _NV) // (_NW * _K), vis_body, 0)

        pltpu.sync_copy(z_h, buf_v)

        def pad_body(i, carry):
            base = (wid + i * _NW) * _K
            pltpu.sync_copy(dstp_h.at[pl.ds(base, _K)], dst_v_ref)
            pltpu.async_copy(buf_v, out_h.at[dst_v_ref], sem_s).wait()
            return carry

        lax.fori_loop(0, trips(n_pad), pad_body, 0)

    return k(embed, proj, tok_t, dst_t, dst_v, dst_p, counts, zrows)


def kernel(visual_features, texts, embed_table, W_proj, b_proj,
           image_token_id, pad_token_id):
    tok_t, dst_t, dst_v, dst_p, counts, attn = _fusion_indices(
        texts, image_token_id, pad_token_id
    )
    vf_flat = visual_features.reshape(-1, visual_features.shape[-1])
    proj = _project(vf_flat, W_proj, b_proj)
    zrows = jnp.zeros((_K, _D), jnp.float32)
    outflat = _sc_fuse(embed_table, proj, tok_t, dst_t, dst_v, dst_p, counts, zrows)
    padded = outflat[:_R].reshape(_B, _MAX_LEN, _D)
    return padded, attn


# speedup vs baseline: 3.4943x; 1.7216x over previous
import functools

import jax
import jax.numpy as jnp
from jax import lax
from jax.experimental import pallas as pl
from jax.experimental.pallas import tpu as pltpu
from jax.experimental.pallas import tpu_sc as plsc

_NC = 2
_NS = 16
_NW = _NC * _NS
_K = 32

_B = 8
_S = 2048
_D = 2048
_NV = 512
_MAX_LEN = 2234
_PADW = 2240
_R = _B * _MAX_LEN


def _fusion_indices(texts, image_token_id, pad_token_id):
    pos = jnp.arange(_S, dtype=jnp.int32)
    toks = texts.astype(jnp.int32)
    L = jnp.sum((toks != pad_token_id).astype(jnp.int32), axis=1)
    valid = pos[None, :] < L[:, None]
    img = (toks == image_token_id) & valid
    n_img = jnp.sum(img.astype(jnp.int32), axis=1)
    vpt = _NV // jnp.maximum(n_img, 1)
    before = jnp.cumsum(img.astype(jnp.int32), axis=1) - img.astype(jnp.int32)
    out_text = pos[None, :] + before * (vpt[:, None] - 1)
    text_act = valid & (~img) & (out_text < _MAX_LEN)
    fa = jnp.argmax(text_act, axis=1)
    dst0 = jnp.take_along_axis(out_text, fa[:, None], axis=1)
    tok0 = jnp.take_along_axis(toks, fa[:, None], axis=1)
    dst_t = jnp.where(text_act, out_text, dst0).reshape(-1)
    tok_t = jnp.where(text_act, toks, tok0).reshape(-1)
    nch_t = (L + _K - 1) // _K

    img_pos = jnp.sort(jnp.where(img, pos[None, :], _S), axis=1)
    vidx = jnp.arange(_NV, dtype=jnp.int32)
    bi = vidx[None, :] // vpt[:, None]
    w = vidx[None, :] - bi * vpt[:, None]
    p_b = jnp.take_along_axis(img_pos, jnp.minimum(bi, _S - 1), axis=1)
    out_vis = p_b + bi * (vpt[:, None] - 1) + w
    nv_b = n_img * vpt
    vis_act = (vidx[None, :] < nv_b[:, None]) & (out_vis < _MAX_LEN)
    src_vis = (jnp.arange(_B, dtype=jnp.int32) * _NV)[:, None] + vidx[None, :]
    dst_v = jnp.where(vis_act, out_vis, out_vis[:, :1])
    src_v = jnp.where(vis_act, src_vis, src_vis[:, :1])
    nch_v = (nv_b + _K - 1) // _K

    length = jnp.minimum(L - n_img + n_img * vpt, _MAX_LEN)
    cols = jnp.arange(_PADW, dtype=jnp.int32)
    padm = (cols[None, :] >= length[:, None]) & (cols[None, :] < _MAX_LEN)
    fillp = jnp.minimum(length, _MAX_LEN - 1)[:, None]
    dst_p = jnp.where(padm, jnp.broadcast_to(cols[None, :], (_B, _PADW)), fillp)
    sbase = (length // _K) * _K
    nch_p = jnp.where(length >= _MAX_LEN, 0, (_PADW - sbase) // _K)

    counts = jnp.concatenate(
        [nch_t, nch_v, nch_p, sbase]).astype(jnp.int32)
    attn = cols[None, :_MAX_LEN] < length[:, None]
    return (tok_t, dst_t, src_v.reshape(-1), dst_v.reshape(-1),
            dst_p.reshape(-1), counts, attn)


def _project(vf_flat, w_proj, b_proj):
    m, k = vf_flat.shape
    n = w_proj.shape[1]
    bm = 512

    def body(a_ref, w_ref, b_ref, o_ref):
        o_ref[...] = (
            jnp.dot(a_ref[...], w_ref[...], preferred_element_type=jnp.float32)
            + b_ref[...]
        )

    return pl.pallas_call(
        body,
        grid=(m // bm,),
        in_specs=[
            pl.BlockSpec((bm, k), lambda i: (i, 0)),
            pl.BlockSpec((k, n), lambda i: (0, 0)),
            pl.BlockSpec((n,), lambda i: (0,)),
        ],
        out_specs=pl.BlockSpec((bm, n), lambda i: (i, 0)),
        out_shape=jax.ShapeDtypeStruct((m, n), jnp.float32),
    )(vf_flat, w_proj, b_proj)


def _sc_fuse(embed, proj, tok_t, dst_t, src_v, dst_v, dst_p, counts, zrows):
    mesh = plsc.VectorSubcoreMesh(
        core_axis_name="c", subcore_axis_name="s", num_cores=_NC, num_subcores=_NS
    )

    @functools.partial(
        pl.kernel,
        out_type=jax.ShapeDtypeStruct((_B, _MAX_LEN, _D), jnp.float32),
        mesh=mesh,
        scratch_types=[
            pltpu.VMEM((32,), jnp.int32),
            pltpu.VMEM((_K,), jnp.int32),
            pltpu.VMEM((_K,), jnp.int32),
            pltpu.VMEM((_K, _D), jnp.float32),
            pltpu.SemaphoreType.DMA,
            pltpu.SemaphoreType.DMA,
        ],
    )
    def k(embed_h, proj_h, tok_h, dstt_h, srcv_h, dstv_h, dstp_h, cnt_h, z_h,
          out_h, cnt_v, idx_v, dst_ref, buf_v, sem_g, sem_s):
        wid = lax.axis_index("s") * _NC + lax.axis_index("c")
        pltpu.sync_copy(cnt_h, cnt_v)
        ca = cnt_v[pl.ds(0, 16)]
        cb = cnt_v[pl.ds(16, 16)]

        def wtrips(nchunks, c0):
            return jnp.maximum(0, (nchunks - c0 + _NW - 1) // _NW)

        for b in range(_B):
            nch = ca[b]
            c0 = (wid + (b * 13) % _NW) & (_NW - 1)

            def text_body(i, carry, b=b, c0=c0):
                base = pl.multiple_of(b * _S + (c0 + i * _NW) * _K, _K)
                pltpu.sync_copy(tok_h.at[pl.ds(base, _K)], idx_v)
                pltpu.sync_copy(dstt_h.at[pl.ds(base, _K)], dst_ref)
                pltpu.async_copy(embed_h.at[idx_v], buf_v, sem_g).wait()
                pltpu.async_copy(buf_v, out_h.at[b].at[dst_ref], sem_s).wait()
                return carry

            lax.fori_loop(0, wtrips(nch, c0), text_body, 0)

        for b in range(_B):
            nch = ca[8 + b]
            c0 = (wid + (b * 16) % _NW) & (_NW - 1)

            def vis_body(i, carry, b=b, c0=c0):
                base = pl.multiple_of(b * _NV + (c0 + i * _NW) * _K, _K)
                pltpu.sync_copy(srcv_h.at[pl.ds(base, _K)], idx_v)
                pltpu.sync_copy(dstv_h.at[pl.ds(base, _K)], dst_ref)
                pltpu.async_copy(proj_h.at[idx_v], buf_v, sem_g).wait()
                pltpu.async_copy(buf_v, out_h.at[b].at[dst_ref], sem_s).wait()
                return carry

            lax.fori_loop(0, wtrips(nch, c0), vis_body, 0)

        pltpu.sync_copy(z_h, buf_v)
        for b in range(_B):
            nch = cb[b]
            sb = cb[8 + b]
            c0 = (wid + (b * 13) % _NW) & (_NW - 1)

            def pad_body(i, carry, b=b, sb=sb, c0=c0):
                base = pl.multiple_of(b * _PADW + sb + (c0 + i * _NW) * _K, _K)
                pltpu.sync_copy(dstp_h.at[pl.ds(base, _K)], dst_ref)
                pltpu.async_copy(buf_v, out_h.at[b].at[dst_ref], sem_s).wait()
                return carry

            lax.fori_loop(0, wtrips(nch, c0), pad_body, 0)

    return k(embed, proj, tok_t, dst_t, src_v, dst_v, dst_p, counts, zrows)


def kernel(visual_features, texts, embed_table, W_proj, b_proj,
           image_token_id, pad_token_id):
    tok_t, dst_t, src_v, dst_v, dst_p, counts, attn = _fusion_indices(
        texts, image_token_id, pad_token_id
    )
    vf_flat = visual_features.reshape(-1, visual_features.shape[-1])
    proj = _project(vf_flat, W_proj, b_proj)
    zrows = jnp.zeros((_K, _D), jnp.float32)
    padded = _sc_fuse(embed_table, proj, tok_t, dst_t, src_v, dst_v, dst_p,
                      counts, zrows)
    return padded, attn


# speedup vs baseline: 5.4796x; 1.5682x over previous
import functools

import jax
import jax.numpy as jnp
from jax import lax
from jax.experimental import pallas as pl
from jax.experimental.pallas import tpu as pltpu
from jax.experimental.pallas import tpu_sc as plsc

_NC = 2
_NS = 16
_NW = _NC * _NS
_K = 32

_B = 8
_S = 2048
_D = 2048
_NV = 512
_MAX_LEN = 2234
_PADW = 2240
_R = _B * _MAX_LEN


def _fusion_indices(texts, image_token_id, pad_token_id):
    pos = jnp.arange(_S, dtype=jnp.int32)
    toks = texts.astype(jnp.int32)
    L = jnp.sum((toks != pad_token_id).astype(jnp.int32), axis=1)
    valid = pos[None, :] < L[:, None]
    img = (toks == image_token_id) & valid
    n_img = jnp.sum(img.astype(jnp.int32), axis=1)
    vpt = _NV // jnp.maximum(n_img, 1)
    before = jnp.cumsum(img.astype(jnp.int32), axis=1) - img.astype(jnp.int32)
    out_text = pos[None, :] + before * (vpt[:, None] - 1)
    text_act = valid & (~img) & (out_text < _MAX_LEN)
    fa = jnp.argmax(text_act, axis=1)
    dst0 = jnp.take_along_axis(out_text, fa[:, None], axis=1)
    tok0 = jnp.take_along_axis(toks, fa[:, None], axis=1)
    dst_t = jnp.where(text_act, out_text, dst0).reshape(-1)
    tok_t = jnp.where(text_act, toks, tok0).reshape(-1)
    nch_t = (L + _K - 1) // _K

    img_pos = jnp.sort(jnp.where(img, pos[None, :], _S), axis=1)
    vidx = jnp.arange(_NV, dtype=jnp.int32)
    bi = vidx[None, :] // vpt[:, None]
    w = vidx[None, :] - bi * vpt[:, None]
    p_b = jnp.take_along_axis(img_pos, jnp.minimum(bi, _S - 1), axis=1)
    out_vis = p_b + bi * (vpt[:, None] - 1) + w
    nv_b = n_img * vpt
    vis_act = (vidx[None, :] < nv_b[:, None]) & (out_vis < _MAX_LEN)
    src_vis = (jnp.arange(_B, dtype=jnp.int32) * _NV)[:, None] + vidx[None, :]
    dst_v = jnp.where(vis_act, out_vis, out_vis[:, :1])
    src_v = jnp.where(vis_act, src_vis, src_vis[:, :1])
    nch_v = (nv_b + _K - 1) // _K

    length = jnp.minimum(L - n_img + n_img * vpt, _MAX_LEN)
    cols = jnp.arange(_PADW, dtype=jnp.int32)
    padm = (cols[None, :] >= length[:, None]) & (cols[None, :] < _MAX_LEN)
    fillp = jnp.minimum(length, _MAX_LEN - 1)[:, None]
    dst_p = jnp.where(padm, jnp.broadcast_to(cols[None, :], (_B, _PADW)), fillp)
    sbase = (length // _K) * _K
    nch_p = jnp.where(length >= _MAX_LEN, 0, (_PADW - sbase) // _K)

    counts = jnp.concatenate(
        [nch_t, nch_v, nch_p, sbase]).astype(jnp.int32)
    attn = cols[None, :_MAX_LEN] < length[:, None]
    return (tok_t, dst_t, src_v.reshape(-1), dst_v.reshape(-1),
            dst_p.reshape(-1), counts, attn)


def _project(vf_flat, w_proj, b_proj):
    m, k = vf_flat.shape
    n = w_proj.shape[1]
    bm = 512

    def body(a_ref, w_ref, b_ref, o_ref):
        o_ref[...] = (
            jnp.dot(a_ref[...], w_ref[...], preferred_element_type=jnp.float32)
            + b_ref[...]
        )

    return pl.pallas_call(
        body,
        grid=(m // bm,),
        in_specs=[
            pl.BlockSpec((bm, k), lambda i: (i, 0)),
            pl.BlockSpec((k, n), lambda i: (0, 0)),
            pl.BlockSpec((n,), lambda i: (0,)),
        ],
        out_specs=pl.BlockSpec((bm, n), lambda i: (i, 0)),
        out_shape=jax.ShapeDtypeStruct((m, n), jnp.float32),
    )(vf_flat, w_proj, b_proj)


def _sc_fuse(embed, proj, tok_t, dst_t, src_v, dst_v, dst_p, counts, zrows):
    mesh = plsc.VectorSubcoreMesh(
        core_axis_name="c", subcore_axis_name="s", num_cores=_NC, num_subcores=_NS
    )

    @functools.partial(
        pl.kernel,
        out_type=jax.ShapeDtypeStruct((_MAX_LEN, _B, _D), jnp.float32),
        mesh=mesh,
        scratch_types=[
            pltpu.VMEM((32,), jnp.int32),
            pltpu.VMEM((_K,), jnp.int32),
            pltpu.VMEM((_K,), jnp.int32),
            pltpu.VMEM((_K, _D), jnp.float32),
            pltpu.SemaphoreType.DMA,
            pltpu.SemaphoreType.DMA,
        ],
    )
    def k(embed_h, proj_h, tok_h, dstt_h, srcv_h, dstv_h, dstp_h, cnt_h, z_h,
          out_h, cnt_v, idx_v, dst_ref, buf_v, sem_g, sem_s):
        wid = lax.axis_index("s") * _NC + lax.axis_index("c")
        pltpu.sync_copy(cnt_h, cnt_v)
        ca = cnt_v[pl.ds(0, 16)]
        cb = cnt_v[pl.ds(16, 16)]

        def wtrips(nchunks, c0):
            return jnp.maximum(0, (nchunks - c0 + _NW - 1) // _NW)

        for b in range(_B):
            nch = ca[b]
            c0 = (wid + (b * 13) % _NW) & (_NW - 1)

            def text_body(i, carry, b=b, c0=c0):
                base = pl.multiple_of(b * _S + (c0 + i * _NW) * _K, _K)
                pltpu.sync_copy(tok_h.at[pl.ds(base, _K)], idx_v)
                pltpu.sync_copy(dstt_h.at[pl.ds(base, _K)], dst_ref)
                pltpu.async_copy(embed_h.at[idx_v], buf_v, sem_g).wait()
                pltpu.async_copy(buf_v, out_h.at[:, b].at[dst_ref], sem_s).wait()
                return carry

            lax.fori_loop(0, wtrips(nch, c0), text_body, 0)

        for b in range(_B):
            nch = ca[8 + b]
            c0 = (wid + (b * 16) % _NW) & (_NW - 1)

            def vis_body(i, carry, b=b, c0=c0):
                base = pl.multiple_of(b * _NV + (c0 + i * _NW) * _K, _K)
                pltpu.sync_copy(srcv_h.at[pl.ds(base, _K)], idx_v)
                pltpu.sync_copy(dstv_h.at[pl.ds(base, _K)], dst_ref)
                pltpu.async_copy(proj_h.at[idx_v], buf_v, sem_g).wait()
                pltpu.async_copy(buf_v, out_h.at[:, b].at[dst_ref], sem_s).wait()
                return carry

            lax.fori_loop(0, wtrips(nch, c0), vis_body, 0)

        pltpu.sync_copy(z_h, buf_v)
        for b in range(_B):
            nch = cb[b]
            sb = cb[8 + b]
            c0 = (wid + (b * 13) % _NW) & (_NW - 1)

            def pad_body(i, carry, b=b, sb=sb, c0=c0):
                base = pl.multiple_of(b * _PADW + sb + (c0 + i * _NW) * _K, _K)
                pltpu.sync_copy(dstp_h.at[pl.ds(base, _K)], dst_ref)
                pltpu.async_copy(buf_v, out_h.at[:, b].at[dst_ref], sem_s).wait()
                return carry

            lax.fori_loop(0, wtrips(nch, c0), pad_body, 0)

    return k(embed, proj, tok_t, dst_t, src_v, dst_v, dst_p, counts, zrows)


def kernel(visual_features, texts, embed_table, W_proj, b_proj,
           image_token_id, pad_token_id):
    tok_t, dst_t, src_v, dst_v, dst_p, counts, attn = _fusion_indices(
        texts, image_token_id, pad_token_id
    )
    vf_flat = visual_features.reshape(-1, visual_features.shape[-1])
    proj = _project(vf_flat, W_proj, b_proj)
    zrows = jnp.zeros((_K, _D), jnp.float32)
    fused = _sc_fuse(embed_table, proj, tok_t, dst_t, src_v, dst_v, dst_p,
                     counts, zrows)
    padded = fused.transpose(1, 0, 2)
    return padded, attn


# speedup vs baseline: 5.5742x; 1.0173x over previous
import functools

import jax
import jax.numpy as jnp
from jax import lax
from jax.experimental import pallas as pl
from jax.experimental.pallas import tpu as pltpu
from jax.experimental.pallas import tpu_sc as plsc

_NC = 2
_NS = 16
_NW = _NC * _NS
_K = 16

_B = 8
_S = 2048
_D = 2048
_NV = 512
_MAX_LEN = 2234
_PADW = 2240
_R = _B * _MAX_LEN


def _fusion_indices(texts, image_token_id, pad_token_id):
    pos = jnp.arange(_S, dtype=jnp.int32)
    toks = texts.astype(jnp.int32)
    L = jnp.sum((toks != pad_token_id).astype(jnp.int32), axis=1)
    valid = pos[None, :] < L[:, None]
    img = (toks == image_token_id) & valid
    n_img = jnp.sum(img.astype(jnp.int32), axis=1)
    vpt = _NV // jnp.maximum(n_img, 1)
    before = jnp.cumsum(img.astype(jnp.int32), axis=1) - img.astype(jnp.int32)
    out_text = pos[None, :] + before * (vpt[:, None] - 1)
    text_act = valid & (~img) & (out_text < _MAX_LEN)
    fa = jnp.argmax(text_act, axis=1)
    dst0 = jnp.take_along_axis(out_text, fa[:, None], axis=1)
    tok0 = jnp.take_along_axis(toks, fa[:, None], axis=1)
    dst_t = jnp.where(text_act, out_text, dst0).reshape(-1)
    tok_t = jnp.where(text_act, toks, tok0).reshape(-1)
    nch_t = (L + _K - 1) // _K

    img_pos = jnp.sort(jnp.where(img, pos[None, :], _S), axis=1)
    vidx = jnp.arange(_NV, dtype=jnp.int32)
    bi = vidx[None, :] // vpt[:, None]
    w = vidx[None, :] - bi * vpt[:, None]
    p_b = jnp.take_along_axis(img_pos, jnp.minimum(bi, _S - 1), axis=1)
    out_vis = p_b + bi * (vpt[:, None] - 1) + w
    nv_b = n_img * vpt
    vis_act = (vidx[None, :] < nv_b[:, None]) & (out_vis < _MAX_LEN)
    src_vis = (jnp.arange(_B, dtype=jnp.int32) * _NV)[:, None] + vidx[None, :]
    dst_v = jnp.where(vis_act, out_vis, out_vis[:, :1])
    src_v = jnp.where(vis_act, src_vis, src_vis[:, :1])
    nch_v = (nv_b + _K - 1) // _K

    length = jnp.minimum(L - n_img + n_img * vpt, _MAX_LEN)
    cols = jnp.arange(_PADW, dtype=jnp.int32)
    padm = (cols[None, :] >= length[:, None]) & (cols[None, :] < _MAX_LEN)
    fillp = jnp.minimum(length, _MAX_LEN - 1)[:, None]
    dst_p = jnp.where(padm, jnp.broadcast_to(cols[None, :], (_B, _PADW)), fillp)
    sbase = (length // _K) * _K
    nch_p = jnp.where(length >= _MAX_LEN, 0, (_PADW - sbase) // _K)

    counts = jnp.concatenate(
        [nch_t, nch_v, nch_p, sbase]).astype(jnp.int32)
    attn = cols[None, :_MAX_LEN] < length[:, None]
    return (tok_t, dst_t, src_v.reshape(-1), dst_v.reshape(-1),
            dst_p.reshape(-1), counts, attn)


def _project(vf_flat, w_proj, b_proj):
    m, k = vf_flat.shape
    n = w_proj.shape[1]
    bm = 512

    def body(a_ref, w_ref, b_ref, o_ref):
        o_ref[...] = (
            jnp.dot(a_ref[...], w_ref[...], preferred_element_type=jnp.float32)
            + b_ref[...]
        )

    return pl.pallas_call(
        body,
        grid=(m // bm,),
        in_specs=[
            pl.BlockSpec((bm, k), lambda i: (i, 0)),
            pl.BlockSpec((k, n), lambda i: (0, 0)),
            pl.BlockSpec((n,), lambda i: (0,)),
        ],
        out_specs=pl.BlockSpec((bm, n), lambda i: (i, 0)),
        out_shape=jax.ShapeDtypeStruct((m, n), jnp.float32),
    )(vf_flat, w_proj, b_proj)


def _sc_fuse(embed, proj, tok_t, dst_t, src_v, dst_v, dst_p, counts, zrows):
    mesh = plsc.VectorSubcoreMesh(
        core_axis_name="c", subcore_axis_name="s", num_cores=_NC, num_subcores=_NS
    )

    @functools.partial(
        pl.kernel,
        out_type=jax.ShapeDtypeStruct((_MAX_LEN, _B, _D), jnp.float32),
        mesh=mesh,
        scratch_types=[
            pltpu.VMEM((32,), jnp.int32),
            [pltpu.VMEM((_K,), jnp.int32)] * 2,
            [pltpu.VMEM((_K,), jnp.int32)] * 2,
            [pltpu.VMEM((_K, _D), jnp.float32)] * 2,
            [pltpu.SemaphoreType.DMA] * 2,
            [pltpu.SemaphoreType.DMA] * 2,
        ],
    )
    def k(embed_h, proj_h, tok_h, dstt_h, srcv_h, dstv_h, dstp_h, cnt_h, z_h,
          out_h, cnt_v, idx_v, dst_ref, buf_v, sem_g, sem_s):
        wid = lax.axis_index("s") * _NC + lax.axis_index("c")
        pltpu.sync_copy(cnt_h, cnt_v)
        ca = cnt_v[pl.ds(0, 16)]
        cb = cnt_v[pl.ds(16, 16)]

        def wtrips(nchunks, c0):
            return jnp.maximum(0, (nchunks - c0 + _NW - 1) // _NW)

        def pipelined(trips, gather_src, gather_wait, load_dst_slice, out_view):
            def pair(j, carry):
                for ph in (0, 1):
                    c = 2 * j + ph

                    @pl.when((c < trips) & (j > 0))
                    def _():
                        pltpu.make_async_copy(
                            buf_v[ph], out_view.at[dst_ref[ph]], sem_s[ph]
                        ).wait()

                    @pl.when(c < trips)
                    def _():
                        load_dst_slice(ph, c)
                        gather_src(ph, c)

                for ph in (0, 1):
                    c = 2 * j + ph

                    @pl.when(c < trips)
                    def _():
                        gather_wait(ph)
                        pltpu.async_copy(
                            buf_v[ph], out_view.at[dst_ref[ph]], sem_s[ph]
                        )

                return carry

            lax.fori_loop(0, (trips + 1) // 2, pair, 0)

            @pl.when(trips >= 1)
            def _():
                pltpu.make_async_copy(
                    buf_v[0], out_view.at[dst_ref[0]], sem_s[0]
                ).wait()

            @pl.when(trips >= 2)
            def _():
                pltpu.make_async_copy(
                    buf_v[1], out_view.at[dst_ref[1]], sem_s[1]
                ).wait()

        for b in range(_B):
            nch = ca[b]
            c0 = (wid + (b * 13) % _NW) & (_NW - 1)
            view = out_h.at[:, b]

            def load_dst(ph, c, b=b, c0=c0):
                base = pl.multiple_of((b * _S) + (c0 + c * _NW) * _K, _K)
                pltpu.sync_copy(dstt_h.at[pl.ds(base, _K)], dst_ref[ph])

            def gather(ph, c, b=b, c0=c0):
                base = pl.multiple_of((b * _S) + (c0 + c * _NW) * _K, _K)
                pltpu.sync_copy(tok_h.at[pl.ds(base, _K)], idx_v[ph])
                pltpu.async_copy(embed_h.at[idx_v[ph]], buf_v[ph], sem_g[ph])

            def gather_wait(ph):
                pltpu.make_async_copy(
                    embed_h.at[idx_v[ph]], buf_v[ph], sem_g[ph]).wait()

            pipelined(wtrips(nch, c0), gather, gather_wait, load_dst, view)

        for b in range(_B):
            nch = ca[8 + b]
            c0 = (wid + (b * 16) % _NW) & (_NW - 1)
            view = out_h.at[:, b]

            def load_dst(ph, c, b=b, c0=c0):
                base = pl.multiple_of((b * _NV) + (c0 + c * _NW) * _K, _K)
                pltpu.sync_copy(dstv_h.at[pl.ds(base, _K)], dst_ref[ph])

            def gather(ph, c, b=b, c0=c0):
                base = pl.multiple_of((b * _NV) + (c0 + c * _NW) * _K, _K)
                pltpu.sync_copy(srcv_h.at[pl.ds(base, _K)], idx_v[ph])
                pltpu.async_copy(proj_h.at[idx_v[ph]], buf_v[ph], sem_g[ph])

            def gather_wait(ph):
                pltpu.make_async_copy(
                    proj_h.at[idx_v[ph]], buf_v[ph], sem_g[ph]).wait()

            pipelined(wtrips(nch, c0), gather, gather_wait, load_dst, view)

        pltpu.sync_copy(z_h, buf_v[0])
        pltpu.sync_copy(z_h, buf_v[1])
        for b in range(_B):
            nch = cb[b]
            sb = cb[8 + b]
            c0 = (wid + (b * 13) % _NW) & (_NW - 1)
            view = out_h.at[:, b]

            def load_dst(ph, c, b=b, sb=sb, c0=c0):
                base = pl.multiple_of(
                    (b * _PADW) + sb + (c0 + c * _NW) * _K, _K)
                pltpu.sync_copy(dstp_h.at[pl.ds(base, _K)], dst_ref[ph])

            def gather(ph, c):
                pass

            def gather_wait(ph):
                pass

            pipelined(wtrips(nch, c0), gather, gather_wait, load_dst, view)

    return k(embed, proj, tok_t, dst_t, src_v, dst_v, dst_p, counts, zrows)


def kernel(visual_features, texts, embed_table, W_proj, b_proj,
           image_token_id, pad_token_id):
    tok_t, dst_t, src_v, dst_v, dst_p, counts, attn = _fusion_indices(
        texts, image_token_id, pad_token_id
    )
    vf_flat = visual_features.reshape(-1, visual_features.shape[-1])
    proj = _project(vf_flat, W_proj, b_proj)
    zrows = jnp.zeros((_K, _D), jnp.float32)
    fused = _sc_fuse(embed_table, proj, tok_t, dst_t, src_v, dst_v, dst_p,
                     counts, zrows)
    padded = fused.transpose(1, 0, 2)
    return padded, attn
